# Initial kernel scaffold; baseline (speedup 1.0000x reference)
import jax
import jax.numpy as jnp
from jax.experimental import pallas as pl


def kernel(output, x_A, y_A, x_B, y_B, ordinal_relation):
    raise NotImplementedError("write your pallas kernel here")


# speedup vs baseline: 1.0424x; 1.0424x over previous
import jax
import jax.numpy as jnp
from jax import lax
from jax.experimental import pallas as pl
from jax.experimental.pallas import tpu as pltpu
from jax.experimental.pallas import tpu_sc as plsc

_NC, _NS, _L = 2, 16, 16
_B, _P, _H, _W = 16, 3000, 256, 256
_PP = 3072
_HALF = _PP // 2
_STEPS = _HALF // _L


def _softplus_steps(map_ref, xa_ref, ya_ref, xb_ref, yb_ref, t_ref):
    def body(j, carry):
        s_vec, c_vec = carry
        off = j * _L
        xa = jnp.clip(xa_ref[pl.ds(off, _L)], 0, _W - 1)
        ya = jnp.clip(ya_ref[pl.ds(off, _L)], 0, _W - 1)
        xb = jnp.clip(xb_ref[pl.ds(off, _L)], 0, _W - 1)
        yb = jnp.clip(yb_ref[pl.ds(off, _L)], 0, _W - 1)
        za = plsc.load_gather(map_ref, [xa * _W + ya])
        zb = plsc.load_gather(map_ref, [xb * _W + yb])
        t = t_ref[pl.ds(off, _L)]
        u = t * (za - zb)
        e = jnp.exp(-jnp.abs(u))
        z = e / (2.0 + e)
        z2 = z * z
        p = 2.0 * z * (1.0 + z2 * (1.0 / 3.0 + z2 * (0.2 + z2 * (1.0 / 7.0 + z2 * (1.0 / 9.0)))))
        val = jnp.maximum(u, 0.0) + p
        m = t != 0.0
        s_vec = s_vec + jnp.where(m, val, 0.0)
        c_vec = c_vec + jnp.where(m, 1.0, 0.0)
        return s_vec, c_vec

    zero = jnp.zeros((_L,), jnp.float32)
    return lax.fori_loop(0, _STEPS, body, (zero, zero))


def _sc_body(flat_hbm, xa_hbm, ya_hbm, xb_hbm, yb_hbm, t_hbm,
             sums_hbm, cnts_hbm,
             map_v, xa_v, ya_v, xb_v, yb_v, t_v, res_s, res_c):
    batch = lax.axis_index("s")
    half = lax.axis_index("c")
    base = half * _HALF
    pltpu.sync_copy(flat_hbm.at[batch], map_v)
    pltpu.sync_copy(xa_hbm.at[batch, pl.ds(base, _HALF)], xa_v)
    pltpu.sync_copy(ya_hbm.at[batch, pl.ds(base, _HALF)], ya_v)
    pltpu.sync_copy(xb_hbm.at[batch, pl.ds(base, _HALF)], xb_v)
    pltpu.sync_copy(yb_hbm.at[batch, pl.ds(base, _HALF)], yb_v)
    pltpu.sync_copy(t_hbm.at[batch, pl.ds(base, _HALF)], t_v)
    s_vec, c_vec = _softplus_steps(map_v, xa_v, ya_v, xb_v, yb_v, t_v)
    res_s[...] = s_vec
    res_c[...] = c_vec
    pltpu.sync_copy(res_s, sums_hbm.at[batch, pl.ds(half * _L, _L)])
    pltpu.sync_copy(res_c, cnts_hbm.at[batch, pl.ds(half * _L, _L)])


@jax.jit
def _sc_partials(flat, xa, ya, xb, yb, t):
    mesh = plsc.VectorSubcoreMesh(core_axis_name="c", subcore_axis_name="s")
    return pl.kernel(
        _sc_body,
        out_type=[
            jax.ShapeDtypeStruct((_B, _NC * _L), jnp.float32),
            jax.ShapeDtypeStruct((_B, _NC * _L), jnp.float32),
        ],
        mesh=mesh,
        compiler_params=pltpu.CompilerParams(needs_layout_passes=False),
        scratch_types=[
            pltpu.VMEM((_H * _W,), jnp.float32),
            pltpu.VMEM((_HALF,), jnp.int32),
            pltpu.VMEM((_HALF,), jnp.int32),
            pltpu.VMEM((_HALF,), jnp.int32),
            pltpu.VMEM((_HALF,), jnp.int32),
            pltpu.VMEM((_HALF,), jnp.float32),
            pltpu.VMEM((_L,), jnp.float32),
            pltpu.VMEM((_L,), jnp.float32),
        ],
    )(flat, xa, ya, xb, yb, t)


def _combine_body(s_ref, c_ref, o_ref):
    s = jnp.sum(s_ref[...], axis=1)
    c = jnp.sum(c_ref[...], axis=1)
    per = s / jnp.maximum(c, 1.0)
    o_ref[...] = (jnp.sum(per) / _B).reshape(1, 1)


@jax.jit
def _combine(sums, cnts):
    return pl.pallas_call(
        _combine_body,
        out_shape=jax.ShapeDtypeStruct((1, 1), jnp.float32),
    )(sums, cnts)


def kernel(output, x_A, y_A, x_B, y_B, ordinal_relation):
    flat = output.reshape(_B, _H * _W).astype(jnp.float32)
    pad = ((0, 0), (0, _PP - _P))
    xa = jnp.pad(x_A.astype(jnp.int32), pad)
    ya = jnp.pad(y_A.astype(jnp.int32), pad)
    xb = jnp.pad(x_B.astype(jnp.int32), pad)
    yb = jnp.pad(y_B.astype(jnp.int32), pad)
    t = jnp.pad(ordinal_relation.astype(jnp.float32), pad)
    sums, cnts = _sc_partials(flat, xa, ya, xb, yb, t)
    return _combine(sums, cnts)[0, 0]
